# Initial kernel scaffold; baseline (speedup 1.0000x reference)
import jax
import jax.numpy as jnp
from jax.experimental import pallas as pl


def kernel(z, u, y, A_vals, eta, A_rows, A_cols):
    raise NotImplementedError("write your pallas kernel here")


# speedup vs baseline: 9.2243x; 9.2243x over previous
import functools

import jax
import jax.numpy as jnp
from jax import lax
from jax.experimental import pallas as pl
from jax.experimental.pallas import tpu as pltpu
from jax.experimental.pallas import tpu_sc as plsc

N = 16384
NNZ = 262144
BATCH = 64
NC = 2
NS = 16
HB = BATCH // NC
NPT = NNZ // NS
CH = 128
NCH = NPT // CH
NG = 4
GC = NCH // NG
SLAB = N // NS
RC = 128
NRC = SLAB // RC

_mesh = plsc.VectorSubcoreMesh(core_axis_name="c", subcore_axis_name="s")


@functools.partial(
    pl.kernel,
    out_type=jax.ShapeDtypeStruct((NC * N, HB), jnp.float32),
    mesh=_mesh,
    compiler_params=pltpu.CompilerParams(use_tc_tiling_on_sc=False),
    scratch_types=[
        pltpu.VMEM_SHARED((N, HB), jnp.float32),
        pltpu.VMEM_SHARED((N, HB), jnp.float32),
        pltpu.VMEM((GC, CH), jnp.int32),
        pltpu.VMEM((GC, CH), jnp.int32),
        pltpu.VMEM((GC, CH), jnp.float32),
        pltpu.VMEM((CH, HB), jnp.float32),
        pltpu.VMEM((RC, HB), jnp.float32),
        pltpu.VMEM((RC, HB), jnp.float32),
        pltpu.VMEM((RC, HB), jnp.float32),
    ],
)
def _rlayer_sc(zf, uf, yf, rows_t, cols_t, vals_t, vale_t,
               out, src_sh, acc_sh, gidx, sidx, valv, gbuf, sa, sb, sc2):
    c = lax.axis_index("c")
    s = lax.axis_index("s")
    base = s * SLAB
    hbase = c * N + base

    def _fill_zero(buf):
        zv = jnp.zeros((16,), jnp.float32)

        def body(r, _):
            buf[r, pl.ds(0, 16)] = zv
            buf[r, pl.ds(16, 16)] = zv
            return 0

        lax.fori_loop(0, RC, body, 0)

    _fill_zero(sb)
    for k in range(NRC):
        r0 = base + k * RC
        h0 = hbase + k * RC
        pltpu.sync_copy(zf.at[pl.ds(h0, RC)], sa)
        pltpu.sync_copy(uf.at[pl.ds(h0, RC)], sc2)

        def mul_body(r, _):
            sa[r, pl.ds(0, 16)] = sa[r, pl.ds(0, 16)] * sc2[r, pl.ds(0, 16)]
            sa[r, pl.ds(16, 16)] = sa[r, pl.ds(16, 16)] * sc2[r, pl.ds(16, 16)]
            return 0

        lax.fori_loop(0, RC, mul_body, 0)
        pltpu.sync_copy(sa, src_sh.at[pl.ds(r0, RC)])
        pltpu.sync_copy(sb, acc_sh.at[pl.ds(r0, RC)])
    plsc.subcore_barrier()

    def spmm_pass(g_hbm, s_hbm, v_hbm):
        def group_body(g, _):
            pltpu.sync_copy(g_hbm.at[s, g], gidx)
            pltpu.sync_copy(s_hbm.at[s, g], sidx)
            pltpu.sync_copy(v_hbm.at[s, g], valv)

            def chunk_body(j, _):
                pltpu.sync_copy(src_sh.at[gidx.at[j]], gbuf)

                def scale_body(q, _):
                    base_cc = q * 16
                    vv = valv[j, pl.ds(base_cc, 16)]
                    for i in range(16):
                        cc = base_cc + i
                        v = vv[i]
                        gbuf[cc, pl.ds(0, 16)] = gbuf[cc, pl.ds(0, 16)] * v
                        gbuf[cc, pl.ds(16, 16)] = gbuf[cc, pl.ds(16, 16)] * v
                    return 0

                lax.fori_loop(0, CH // 16, scale_body, 0)
                pltpu.sync_copy(gbuf, acc_sh.at[sidx.at[j]], add=True)
                return 0

            lax.fori_loop(0, GC, chunk_body, 0)
            return 0

        lax.fori_loop(0, NG, group_body, 0)

    spmm_pass(cols_t, rows_t, vals_t)
    plsc.subcore_barrier()

    _fill_zero(sc2)
    for k in range(NRC):
        r0 = base + k * RC
        h0 = hbase + k * RC
        pltpu.sync_copy(acc_sh.at[pl.ds(r0, RC)], sa)
        pltpu.sync_copy(yf.at[pl.ds(h0, RC)], sb)

        def sub_body(r, _):
            sb[r, pl.ds(0, 16)] = sb[r, pl.ds(0, 16)] - sa[r, pl.ds(0, 16)]
            sb[r, pl.ds(16, 16)] = sb[r, pl.ds(16, 16)] - sa[r, pl.ds(16, 16)]
            return 0

        lax.fori_loop(0, RC, sub_body, 0)
        pltpu.sync_copy(sb, src_sh.at[pl.ds(r0, RC)])
        pltpu.sync_copy(sc2, acc_sh.at[pl.ds(r0, RC)])
    plsc.subcore_barrier()

    spmm_pass(rows_t, cols_t, vale_t)
    plsc.subcore_barrier()

    for k in range(NRC):
        r0 = base + k * RC
        h0 = hbase + k * RC
        pltpu.sync_copy(acc_sh.at[pl.ds(r0, RC)], sa)
        pltpu.sync_copy(zf.at[pl.ds(h0, RC)], sb)
        pltpu.sync_copy(uf.at[pl.ds(h0, RC)], sc2)

        def fin_body(r, _):
            sb[r, pl.ds(0, 16)] = (
                sb[r, pl.ds(0, 16)] - sc2[r, pl.ds(0, 16)] * sa[r, pl.ds(0, 16)]
            )
            sb[r, pl.ds(16, 16)] = (
                sb[r, pl.ds(16, 16)] - sc2[r, pl.ds(16, 16)] * sa[r, pl.ds(16, 16)]
            )
            return 0

        lax.fori_loop(0, RC, fin_body, 0)
        pltpu.sync_copy(sb, out.at[pl.ds(h0, RC)])


def _to_sc_layout(x):
    return x.reshape(NC, HB, N).transpose(0, 2, 1).reshape(NC * N, HB)


def kernel(z, u, y, A_vals, eta, A_rows, A_cols):
    zf = _to_sc_layout(z)
    uf = _to_sc_layout(u)
    yf = _to_sc_layout(y)
    rows_t = A_rows.reshape(NS, NG, GC, CH)
    cols_t = A_cols.reshape(NS, NG, GC, CH)
    vals_t = A_vals.reshape(NS, NG, GC, CH)
    vale_t = (A_vals * eta).reshape(NS, NG, GC, CH)
    o = _rlayer_sc(zf, uf, yf, rows_t, cols_t, vals_t, vale_t)
    return o.reshape(NC, N, HB).transpose(0, 2, 1).reshape(BATCH, N)


# speedup vs baseline: 12.4523x; 1.3499x over previous
import functools

import jax
import jax.numpy as jnp
from jax import lax
from jax.experimental import pallas as pl
from jax.experimental.pallas import tpu as pltpu
from jax.experimental.pallas import tpu_sc as plsc

N = 16384
NNZ = 262144
BATCH = 64
NC = 2
NS = 16
HB = BATCH // NC
NPT = NNZ // NS
CH = 128
NCH = NPT // CH
NG = 4
GC = NCH // NG
DEPTH = 4
SLAB = N // NS
RC = 128
NRC = SLAB // RC

_mesh = plsc.VectorSubcoreMesh(core_axis_name="c", subcore_axis_name="s")


@functools.partial(
    pl.kernel,
    out_type=jax.ShapeDtypeStruct((NC * N, HB), jnp.float32),
    mesh=_mesh,
    compiler_params=pltpu.CompilerParams(use_tc_tiling_on_sc=False),
    scratch_types=[
        pltpu.VMEM_SHARED((N, HB), jnp.float32),
        pltpu.VMEM_SHARED((N, HB), jnp.float32),
        pltpu.VMEM((2, GC, CH), jnp.int32),
        pltpu.VMEM((2, GC, CH), jnp.int32),
        pltpu.VMEM((2, GC, CH), jnp.float32),
        pltpu.VMEM((DEPTH, CH, HB), jnp.float32),
        pltpu.VMEM((RC, HB), jnp.float32),
        pltpu.VMEM((RC, HB), jnp.float32),
        pltpu.VMEM((RC, HB), jnp.float32),
        pltpu.SemaphoreType.DMA((DEPTH,)),
        pltpu.SemaphoreType.DMA((DEPTH,)),
        pltpu.SemaphoreType.DMA,
    ],
)
def _rlayer_sc(zf, uf, yf, rows_t, cols_t, vals_t, vale_t,
               out, src_sh, acc_sh, gidx, sidx, valv, gbuf, sa, sb, sc2,
               gsem, ssem, isem):
    c = lax.axis_index("c")
    s = lax.axis_index("s")
    base = s * SLAB
    hbase = c * N + base

    def _fill_zero(buf):
        zv = jnp.zeros((16,), jnp.float32)

        def body(r, _):
            buf[r, pl.ds(0, 16)] = zv
            buf[r, pl.ds(16, 16)] = zv
            return 0

        lax.fori_loop(0, RC, body, 0)

    _fill_zero(sb)
    for k in range(NRC):
        r0 = base + k * RC
        h0 = hbase + k * RC
        pltpu.sync_copy(zf.at[pl.ds(h0, RC)], sa)
        pltpu.sync_copy(uf.at[pl.ds(h0, RC)], sc2)

        def mul_body(r, _):
            sa[r, pl.ds(0, 16)] = sa[r, pl.ds(0, 16)] * sc2[r, pl.ds(0, 16)]
            sa[r, pl.ds(16, 16)] = sa[r, pl.ds(16, 16)] * sc2[r, pl.ds(16, 16)]
            return 0

        lax.fori_loop(0, RC, mul_body, 0)
        pltpu.sync_copy(sa, src_sh.at[pl.ds(r0, RC)])
        pltpu.sync_copy(sb, acc_sh.at[pl.ds(r0, RC)])
    plsc.subcore_barrier()

    def spmm_pass(g_hbm, s_hbm, v_hbm):
        def wait_buf(sem_slice, dst):
            pltpu.make_async_copy(zf.at[pl.ds(0, CH)], dst, sem_slice).wait()

        pltpu.async_copy(g_hbm.at[s, 0], gidx.at[0], isem)
        pltpu.async_copy(s_hbm.at[s, 0], sidx.at[0], isem)
        pltpu.async_copy(v_hbm.at[s, 0], valv.at[0], isem)

        def group_body(g, _):
            gset = lax.rem(g, 2)
            pltpu.make_async_copy(g_hbm.at[s, g], gidx.at[gset], isem).wait()
            pltpu.make_async_copy(s_hbm.at[s, g], sidx.at[gset], isem).wait()
            pltpu.make_async_copy(v_hbm.at[s, g], valv.at[gset], isem).wait()

            @pl.when(g < NG - 1)
            def _():
                nset = lax.rem(g + 1, 2)
                pltpu.async_copy(g_hbm.at[s, g + 1], gidx.at[nset], isem)
                pltpu.async_copy(s_hbm.at[s, g + 1], sidx.at[nset], isem)
                pltpu.async_copy(v_hbm.at[s, g + 1], valv.at[nset], isem)

            for p in range(DEPTH - 1):
                pltpu.async_copy(
                    src_sh.at[gidx.at[gset, p]], gbuf.at[p], gsem.at[p]
                )

            def slot(j, p):
                wait_buf(gsem.at[p], gbuf.at[p])

                def scale_body(q, _):
                    base_cc = q * 16
                    vv = valv[gset, j, pl.ds(base_cc, 16)]
                    for i in range(16):
                        v = vv[i]
                        cc = base_cc + i
                        gbuf[p, cc, pl.ds(0, 16)] = gbuf[p, cc, pl.ds(0, 16)] * v
                        gbuf[p, cc, pl.ds(16, 16)] = gbuf[p, cc, pl.ds(16, 16)] * v
                    return 0

                lax.fori_loop(0, CH // 16, scale_body, 0)
                pltpu.async_copy(
                    gbuf.at[p], acc_sh.at[sidx.at[gset, j]], ssem.at[p],
                    add=True,
                )
                nxt = (p + DEPTH - 1) % DEPTH

                @pl.when(j >= 1)
                def _():
                    wait_buf(ssem.at[nxt], gbuf.at[nxt])

                @pl.when(j + DEPTH - 1 < GC)
                def _():
                    pltpu.async_copy(
                        src_sh.at[gidx.at[gset, j + DEPTH - 1]],
                        gbuf.at[nxt], gsem.at[nxt],
                    )

            def ring_body(kk, _):
                for p in range(DEPTH):
                    slot(kk * DEPTH + p, p)
                return 0

            lax.fori_loop(0, GC // DEPTH, ring_body, 0)
            wait_buf(ssem.at[(GC - 1) % DEPTH], gbuf.at[(GC - 1) % DEPTH])
            return 0

        lax.fori_loop(0, NG, group_body, 0)

    spmm_pass(cols_t, rows_t, vals_t)
    plsc.subcore_barrier()

    _fill_zero(sc2)
    for k in range(NRC):
        r0 = base + k * RC
        h0 = hbase + k * RC
        pltpu.sync_copy(acc_sh.at[pl.ds(r0, RC)], sa)
        pltpu.sync_copy(yf.at[pl.ds(h0, RC)], sb)

        def sub_body(r, _):
            sb[r, pl.ds(0, 16)] = sb[r, pl.ds(0, 16)] - sa[r, pl.ds(0, 16)]
            sb[r, pl.ds(16, 16)] = sb[r, pl.ds(16, 16)] - sa[r, pl.ds(16, 16)]
            return 0

        lax.fori_loop(0, RC, sub_body, 0)
        pltpu.sync_copy(sb, src_sh.at[pl.ds(r0, RC)])
        pltpu.sync_copy(sc2, acc_sh.at[pl.ds(r0, RC)])
    plsc.subcore_barrier()

    spmm_pass(rows_t, cols_t, vale_t)
    plsc.subcore_barrier()

    for k in range(NRC):
        r0 = base + k * RC
        h0 = hbase + k * RC
        pltpu.sync_copy(acc_sh.at[pl.ds(r0, RC)], sa)
        pltpu.sync_copy(zf.at[pl.ds(h0, RC)], sb)
        pltpu.sync_copy(uf.at[pl.ds(h0, RC)], sc2)

        def fin_body(r, _):
            sb[r, pl.ds(0, 16)] = (
                sb[r, pl.ds(0, 16)] - sc2[r, pl.ds(0, 16)] * sa[r, pl.ds(0, 16)]
            )
            sb[r, pl.ds(16, 16)] = (
                sb[r, pl.ds(16, 16)] - sc2[r, pl.ds(16, 16)] * sa[r, pl.ds(16, 16)]
            )
            return 0

        lax.fori_loop(0, RC, fin_body, 0)
        pltpu.sync_copy(sb, out.at[pl.ds(h0, RC)])


def _to_sc_layout(x):
    return x.reshape(NC, HB, N).transpose(0, 2, 1).reshape(NC * N, HB)


def kernel(z, u, y, A_vals, eta, A_rows, A_cols):
    zf = _to_sc_layout(z)
    uf = _to_sc_layout(u)
    yf = _to_sc_layout(y)
    rows_t = A_rows.reshape(NS, NG, GC, CH)
    cols_t = A_cols.reshape(NS, NG, GC, CH)
    vals_t = A_vals.reshape(NS, NG, GC, CH)
    vale_t = (A_vals * eta).reshape(NS, NG, GC, CH)
    o = _rlayer_sc(zf, uf, yf, rows_t, cols_t, vals_t, vale_t)
    return o.reshape(NC, N, HB).transpose(0, 2, 1).reshape(BATCH, N)


# speedup vs baseline: 12.5745x; 1.0098x over previous
import functools

import jax
import jax.numpy as jnp
from jax import lax
from jax.experimental import pallas as pl
from jax.experimental.pallas import tpu as pltpu
from jax.experimental.pallas import tpu_sc as plsc

N = 16384
NNZ = 262144
BATCH = 64
NC = 2
NS = 16
HB = BATCH // NC
NPT = NNZ // NS
CH = 128
NCH = NPT // CH
NG = 4
GC = NCH // NG
DEPTH = 4
SLAB = N // NS
RC = 128
NRC = SLAB // RC

_mesh = plsc.VectorSubcoreMesh(core_axis_name="c", subcore_axis_name="s")


@functools.partial(
    pl.kernel,
    out_type=jax.ShapeDtypeStruct((NC * N, HB), jnp.float32),
    mesh=_mesh,
    compiler_params=pltpu.CompilerParams(use_tc_tiling_on_sc=False),
    scratch_types=[
        pltpu.VMEM_SHARED((N, HB), jnp.float32),
        pltpu.VMEM_SHARED((N, HB), jnp.float32),
        pltpu.VMEM((2, GC, CH), jnp.int32),
        pltpu.VMEM((2, GC, CH), jnp.int32),
        pltpu.VMEM((2, GC, CH), jnp.float32),
        pltpu.VMEM((DEPTH, CH, HB), jnp.float32),
        pltpu.VMEM((RC, HB), jnp.float32),
        pltpu.VMEM((RC, HB), jnp.float32),
        pltpu.VMEM((RC, HB), jnp.float32),
        pltpu.SemaphoreType.DMA((DEPTH,)),
        pltpu.SemaphoreType.DMA((DEPTH,)),
        pltpu.SemaphoreType.DMA,
    ],
)
def _rlayer_sc(zf, uf, yf, rows_t, cols_t, vals_t, vale_t,
               out, src_sh, acc_sh, gidx, sidx, valv, gbuf, sa, sb, sc2,
               gsem, ssem, isem):
    c = lax.axis_index("c")
    s = lax.axis_index("s")
    base = s * SLAB
    hbase = c * N + base

    def _fill_zero(buf):
        zv = jnp.zeros((16,), jnp.float32)

        def body(r, _):
            buf[r, pl.ds(0, 16)] = zv
            buf[r, pl.ds(16, 16)] = zv
            return 0

        lax.fori_loop(0, RC, body, 0)

    _fill_zero(sb)
    for k in range(NRC):
        r0 = base + k * RC
        h0 = hbase + k * RC
        pltpu.sync_copy(zf.at[pl.ds(h0, RC)], sa)
        pltpu.sync_copy(uf.at[pl.ds(h0, RC)], sc2)

        def mul_body(r, _):
            sa[r, pl.ds(0, 16)] = sa[r, pl.ds(0, 16)] * sc2[r, pl.ds(0, 16)]
            sa[r, pl.ds(16, 16)] = sa[r, pl.ds(16, 16)] * sc2[r, pl.ds(16, 16)]
            return 0

        lax.fori_loop(0, RC, mul_body, 0)
        pltpu.sync_copy(sa, src_sh.at[pl.ds(r0, RC)])
        pltpu.sync_copy(sb, acc_sh.at[pl.ds(r0, RC)])
    plsc.subcore_barrier()

    def spmm_pass(g_hbm, s_hbm, v_hbm):
        def wait_buf(sem_slice, dst):
            pltpu.make_async_copy(zf.at[pl.ds(0, CH)], dst, sem_slice).wait()

        pltpu.async_copy(g_hbm.at[s, 0], gidx.at[0], isem)
        pltpu.async_copy(s_hbm.at[s, 0], sidx.at[0], isem)
        pltpu.async_copy(v_hbm.at[s, 0], valv.at[0], isem)

        def group_body(g, _):
            gset = lax.rem(g, 2)
            pltpu.make_async_copy(g_hbm.at[s, g], gidx.at[gset], isem).wait()
            pltpu.make_async_copy(s_hbm.at[s, g], sidx.at[gset], isem).wait()
            pltpu.make_async_copy(v_hbm.at[s, g], valv.at[gset], isem).wait()

            @pl.when(g < NG - 1)
            def _():
                nset = lax.rem(g + 1, 2)
                pltpu.async_copy(g_hbm.at[s, g + 1], gidx.at[nset], isem)
                pltpu.async_copy(s_hbm.at[s, g + 1], sidx.at[nset], isem)
                pltpu.async_copy(v_hbm.at[s, g + 1], valv.at[nset], isem)

            for p in range(DEPTH - 1):
                pltpu.async_copy(
                    src_sh.at[gidx.at[gset, p]], gbuf.at[p], gsem.at[p]
                )

            def slot(j, p):
                wait_buf(gsem.at[p], gbuf.at[p])

                @plsc.parallel_loop(0, CH // 16, 1, unroll=2)
                def scale_body(q):
                    base_cc = q * 16
                    vv = valv[gset, j, pl.ds(base_cc, 16)]
                    for i in range(16):
                        v = vv[i]
                        cc = base_cc + i
                        gbuf[p, cc, pl.ds(0, 16)] = gbuf[p, cc, pl.ds(0, 16)] * v
                        gbuf[p, cc, pl.ds(16, 16)] = gbuf[p, cc, pl.ds(16, 16)] * v
                pltpu.async_copy(
                    gbuf.at[p], acc_sh.at[sidx.at[gset, j]], ssem.at[p],
                    add=True,
                )
                nxt = (p + DEPTH - 1) % DEPTH

                @pl.when(j >= 1)
                def _():
                    wait_buf(ssem.at[nxt], gbuf.at[nxt])

                @pl.when(j + DEPTH - 1 < GC)
                def _():
                    pltpu.async_copy(
                        src_sh.at[gidx.at[gset, j + DEPTH - 1]],
                        gbuf.at[nxt], gsem.at[nxt],
                    )

            def ring_body(kk, _):
                for p in range(DEPTH):
                    slot(kk * DEPTH + p, p)
                return 0

            lax.fori_loop(0, GC // DEPTH, ring_body, 0)
            wait_buf(ssem.at[(GC - 1) % DEPTH], gbuf.at[(GC - 1) % DEPTH])
            return 0

        lax.fori_loop(0, NG, group_body, 0)

    spmm_pass(cols_t, rows_t, vals_t)
    plsc.subcore_barrier()

    _fill_zero(sc2)
    for k in range(NRC):
        r0 = base + k * RC
        h0 = hbase + k * RC
        pltpu.sync_copy(acc_sh.at[pl.ds(r0, RC)], sa)
        pltpu.sync_copy(yf.at[pl.ds(h0, RC)], sb)

        def sub_body(r, _):
            sb[r, pl.ds(0, 16)] = sb[r, pl.ds(0, 16)] - sa[r, pl.ds(0, 16)]
            sb[r, pl.ds(16, 16)] = sb[r, pl.ds(16, 16)] - sa[r, pl.ds(16, 16)]
            return 0

        lax.fori_loop(0, RC, sub_body, 0)
        pltpu.sync_copy(sb, src_sh.at[pl.ds(r0, RC)])
        pltpu.sync_copy(sc2, acc_sh.at[pl.ds(r0, RC)])
    plsc.subcore_barrier()

    spmm_pass(rows_t, cols_t, vale_t)
    plsc.subcore_barrier()

    for k in range(NRC):
        r0 = base + k * RC
        h0 = hbase + k * RC
        pltpu.sync_copy(acc_sh.at[pl.ds(r0, RC)], sa)
        pltpu.sync_copy(zf.at[pl.ds(h0, RC)], sb)
        pltpu.sync_copy(uf.at[pl.ds(h0, RC)], sc2)

        def fin_body(r, _):
            sb[r, pl.ds(0, 16)] = (
                sb[r, pl.ds(0, 16)] - sc2[r, pl.ds(0, 16)] * sa[r, pl.ds(0, 16)]
            )
            sb[r, pl.ds(16, 16)] = (
                sb[r, pl.ds(16, 16)] - sc2[r, pl.ds(16, 16)] * sa[r, pl.ds(16, 16)]
            )
            return 0

        lax.fori_loop(0, RC, fin_body, 0)
        pltpu.sync_copy(sb, out.at[pl.ds(h0, RC)])


def _to_sc_layout(x):
    return x.reshape(NC, HB, N).transpose(0, 2, 1).reshape(NC * N, HB)


def kernel(z, u, y, A_vals, eta, A_rows, A_cols):
    zf = _to_sc_layout(z)
    uf = _to_sc_layout(u)
    yf = _to_sc_layout(y)
    rows_t = A_rows.reshape(NS, NG, GC, CH)
    cols_t = A_cols.reshape(NS, NG, GC, CH)
    vals_t = A_vals.reshape(NS, NG, GC, CH)
    vale_t = (A_vals * eta).reshape(NS, NG, GC, CH)
    o = _rlayer_sc(zf, uf, yf, rows_t, cols_t, vals_t, vale_t)
    return o.reshape(NC, N, HB).transpose(0, 2, 1).reshape(BATCH, N)
